# baseline (speedup 1.0000x reference)
import jax
import jax.numpy as jnp
from jax.experimental import pallas as pl
from jax.experimental.pallas import tpu as pltpu

_B, _T, _D, _H = 1024, 200, 64, 64
_S = 8
_P = _B // 2


def _lstm_body(x_ref, w_ref, b_ref, out_ref, h_ref, c_ref):
    t = pl.program_id(0)

    @pl.when(t == 0)
    def _init():
        h_ref[...] = jnp.zeros_like(h_ref)
        c_ref[...] = jnp.zeros_like(c_ref)

    wxh = w_ref[...]
    b = b_ref[0:1, :]
    dn = (((1,), (0,)), ((), ()))
    w2 = 2 * _H

    h = h_ref[...]
    c = c_ref[...]
    houts = []
    for s in range(_S):
        xa = x_ref[0:_P, s * _D:(s + 1) * _D]
        xb = x_ref[_P:_B, s * _D:(s + 1) * _D]
        lhs = jnp.concatenate([xa, xb, h.astype(jnp.bfloat16)], axis=-1)
        gates = jax.lax.dot_general(
            lhs, wxh, dn, preferred_element_type=jnp.float32) + b
        ti = jnp.tanh(gates[:, 0 * w2:1 * w2])
        tf = jnp.tanh(gates[:, 1 * w2:2 * w2])
        tg = jnp.tanh(gates[:, 2 * w2:3 * w2])
        to = jnp.tanh(gates[:, 3 * w2:4 * w2])
        c = (tf * 0.5 + 0.5) * c + (ti * 0.5 + 0.5) * tg
        h = (to * 0.5 + 0.5) * jnp.tanh(c)
        houts.append(h)
    tr = jnp.transpose(jnp.stack(houts, axis=0), (1, 0, 2))
    out_ref[0:_P] = tr[:, :, :_H]
    out_ref[_P:_B] = tr[:, :, _H:]
    h_ref[...] = h
    c_ref[...] = c


def _block_diag(w):
    d = w.shape[0]
    w4 = w.reshape(d, 4, _H)
    out = jnp.zeros((2 * d, 4, 2, _H), dtype=w.dtype)
    out = out.at[:d, :, 0, :].set(w4)
    out = out.at[d:, :, 1, :].set(w4)
    return out.reshape(2 * d, 8 * _H)


def kernel(x, W_ih, W_hh, b_ih, b_hh):
    scale = jnp.concatenate([
        jnp.full((2 * _H,), 0.5, jnp.float32),
        jnp.ones((_H,), jnp.float32),
        jnp.full((_H,), 0.5, jnp.float32),
    ])
    wx_bd = _block_diag(W_ih.T * scale[None, :]).astype(jnp.bfloat16)
    wh_bd = _block_diag(W_hh.T * scale[None, :]).astype(jnp.bfloat16)
    wxh = jnp.concatenate([wx_bd, wh_bd], axis=0)
    b4 = ((b_ih + b_hh) * scale).reshape(4, _H)
    b_p = jnp.concatenate([b4, b4], axis=-1).reshape(8 * _H)
    b_row = jnp.broadcast_to(b_p[None, :], (8, 8 * _H))

    x2 = x.reshape(_B, _T * _D).astype(jnp.bfloat16)

    grid = (_T // _S,)

    out = pl.pallas_call(
        _lstm_body,
        grid=grid,
        in_specs=[
            pl.BlockSpec((_B, _S * _D), lambda t: (0, t)),
            pl.BlockSpec((2 * _D + 2 * _H, 8 * _H), lambda t: (0, 0)),
            pl.BlockSpec((8, 8 * _H), lambda t: (0, 0)),
        ],
        out_specs=pl.BlockSpec((_B, _S, _H), lambda t: (0, t, 0)),
        out_shape=jax.ShapeDtypeStruct((_B, _T, _H), jnp.float32),
        scratch_shapes=[
            pltpu.VMEM((_P, 2 * _H), jnp.float32),
            pltpu.VMEM((_P, 2 * _H), jnp.float32),
        ],
        compiler_params=pltpu.CompilerParams(
            dimension_semantics=("arbitrary",),
        ),
    )(x2, wxh, b_row)

    return out


# speedup vs baseline: 1.3285x; 1.3285x over previous
import jax
import jax.numpy as jnp
from jax.experimental import pallas as pl
from jax.experimental.pallas import tpu as pltpu

_B, _T, _D, _H = 1024, 200, 64, 64
_S = 20
_P = _B // 2


def _lstm_body(x_ref, w_ref, b_ref, out_ref, h_ref, c_ref):
    t = pl.program_id(0)

    @pl.when(t == 0)
    def _init():
        h_ref[...] = jnp.zeros_like(h_ref)
        c_ref[...] = jnp.zeros_like(c_ref)

    wxh = w_ref[...]
    b = b_ref[0:1, :]
    dn = (((1,), (0,)), ((), ()))
    w2 = 2 * _H

    h = h_ref[...]
    c = c_ref[...]
    for s in range(_S):
        xa = x_ref[0:_P, s * _D:(s + 1) * _D]
        xb = x_ref[_P:_B, s * _D:(s + 1) * _D]
        lhs = jnp.concatenate([xa, xb, h.astype(jnp.bfloat16)], axis=-1)
        gates = jax.lax.dot_general(
            lhs, wxh, dn, preferred_element_type=jnp.float32) + b
        ti = jnp.tanh(gates[:, 0 * w2:1 * w2])
        tf = jnp.tanh(gates[:, 1 * w2:2 * w2])
        tg = jnp.tanh(gates[:, 2 * w2:3 * w2])
        to = jnp.tanh(gates[:, 3 * w2:4 * w2])
        c = (tf * 0.5 + 0.5) * c + (ti * 0.5 + 0.5) * tg
        h = (to * 0.5 + 0.5) * jnp.tanh(c)
        hb = h.astype(jnp.bfloat16)
        out_ref[0:_P, s * _H:(s + 1) * _H] = hb[:, :_H]
        out_ref[_P:_B, s * _H:(s + 1) * _H] = hb[:, _H:]
    h_ref[...] = h
    c_ref[...] = c


def _block_diag(w):
    d = w.shape[0]
    w4 = w.reshape(d, 4, _H)
    out = jnp.zeros((2 * d, 4, 2, _H), dtype=w.dtype)
    out = out.at[:d, :, 0, :].set(w4)
    out = out.at[d:, :, 1, :].set(w4)
    return out.reshape(2 * d, 8 * _H)


def kernel(x, W_ih, W_hh, b_ih, b_hh):
    scale = jnp.concatenate([
        jnp.full((2 * _H,), 0.5, jnp.float32),
        jnp.ones((_H,), jnp.float32),
        jnp.full((_H,), 0.5, jnp.float32),
    ])
    wx_bd = _block_diag(W_ih.T * scale[None, :]).astype(jnp.bfloat16)
    wh_bd = _block_diag(W_hh.T * scale[None, :]).astype(jnp.bfloat16)
    wxh = jnp.concatenate([wx_bd, wh_bd], axis=0)
    b4 = ((b_ih + b_hh) * scale).reshape(4, _H)
    b_p = jnp.concatenate([b4, b4], axis=-1).reshape(8 * _H)
    b_row = jnp.broadcast_to(b_p[None, :], (8, 8 * _H))

    x2 = x.reshape(_B, _T * _D).astype(jnp.bfloat16)

    grid = (_T // _S,)

    out = pl.pallas_call(
        _lstm_body,
        grid=grid,
        in_specs=[
            pl.BlockSpec((_B, _S * _D), lambda t: (0, t)),
            pl.BlockSpec((2 * _D + 2 * _H, 8 * _H), lambda t: (0, 0)),
            pl.BlockSpec((8, 8 * _H), lambda t: (0, 0)),
        ],
        out_specs=pl.BlockSpec((_B, _S * _H), lambda t: (0, t)),
        out_shape=jax.ShapeDtypeStruct((_B, _T * _H), jnp.bfloat16),
        scratch_shapes=[
            pltpu.VMEM((_P, 2 * _H), jnp.float32),
            pltpu.VMEM((_P, 2 * _H), jnp.float32),
        ],
        compiler_params=pltpu.CompilerParams(
            dimension_semantics=("arbitrary",),
        ),
    )(x2, wxh, b_row)

    return out.reshape(_B, _T, _H).astype(jnp.float32)
